# baseline (speedup 1.0000x reference)
import jax
import jax.numpy as jnp
from jax.experimental import pallas as pl

D_MODEL_ = 32
NUM_EXPERTS_ = 128
N_TOKENS_ = 8192
D_FF_ = 4 * D_MODEL_
BLK_ = 1024


def _combine_kernel(xg_ref, gw_ref, gb_ref, ewt_ref, eb_ref, mt_ref, b2_ref):
    logits = jnp.dot(xg_ref[...], gw_ref[...].T,
                     preferred_element_type=jnp.float32) + gb_ref[...]
    w = jax.nn.softmax(logits, axis=-1)
    mt_ref[...] = jnp.sum(ewt_ref[...] * w[None, :, :], axis=1)
    b2_ref[...] = jnp.sum(w * eb_ref[...], axis=0, keepdims=True)


def _matmul_kernel(x_ref, mt_ref, b2_ref, o_ref):
    o_ref[...] = jnp.dot(x_ref[...], mt_ref[...],
                         preferred_element_type=jnp.float32) + b2_ref[...]


def kernel(x, gate_w, gate_b, expert_w, expert_b):
    ewt = jnp.transpose(expert_w, (2, 0, 1))
    gb = gate_b.reshape(1, NUM_EXPERTS_)
    mt, b2 = pl.pallas_call(
        _combine_kernel,
        out_shape=(
            jax.ShapeDtypeStruct((D_MODEL_, NUM_EXPERTS_), jnp.float32),
            jax.ShapeDtypeStruct((1, NUM_EXPERTS_), jnp.float32),
        ),
    )(x[:NUM_EXPERTS_], gate_w, gb, ewt, expert_b)
    return pl.pallas_call(
        _matmul_kernel,
        grid=(N_TOKENS_ // BLK_,),
        in_specs=[
            pl.BlockSpec((BLK_, D_MODEL_), lambda i: (i, 0)),
            pl.BlockSpec((D_MODEL_, NUM_EXPERTS_), lambda i: (0, 0)),
            pl.BlockSpec((1, NUM_EXPERTS_), lambda i: (0, 0)),
        ],
        out_specs=pl.BlockSpec((BLK_, NUM_EXPERTS_), lambda i: (i, 0)),
        out_shape=jax.ShapeDtypeStruct((N_TOKENS_, NUM_EXPERTS_), jnp.float32),
    )(x, mt, b2)
